# Initial kernel scaffold; baseline (speedup 1.0000x reference)
import functools

import jax
import jax.numpy as jnp
from jax import lax
from jax.experimental import pallas as pl
from jax.experimental.pallas import tpu as pltpu
from jax.experimental.pallas import tpu_sc as plsc

NC = 2
NS = 16
LANES = 16


@functools.partial(jax.jit, static_argnames=("n", "e"))
def _sc_count(dst, *, n, e):
    nw = NC * NS
    epw = e // nw
    blk = 80
    nblk = epw // blk
    rpt = n // NS
    zr = 25

    mesh = plsc.VectorSubcoreMesh(
        core_axis_name="c", subcore_axis_name="s",
        num_cores=NC, num_subcores=NS)

    @functools.partial(
        pl.kernel,
        out_type=jax.ShapeDtypeStruct((NC, n, LANES), jnp.float32),
        mesh=mesh,
        scratch_types=[
            pltpu.VMEM((blk,), jnp.int32),
            pltpu.VMEM((blk, LANES), jnp.float32),
            pltpu.VMEM((zr, LANES), jnp.float32),
            pltpu.VMEM((rpt, LANES), jnp.float32),
            pltpu.VMEM_SHARED((n, LANES), jnp.float32),
        ],
    )
    def k(dst_hbm, out_hbm, dst_v, ones_v, zbuf, obuf, acc):
        c = lax.axis_index("c")
        s = lax.axis_index("s")
        wid = s * NC + c
        one = jnp.ones((LANES,), jnp.float32)
        zero = jnp.zeros((LANES,), jnp.float32)
        for r in range(blk):
            ones_v[r, :] = one
        for r in range(zr):
            zbuf[r, :] = zero
        row0 = s * rpt

        def zb(i, _):
            pltpu.sync_copy(zbuf, acc.at[pl.ds(row0 + i * zr, zr)])
            return 0

        lax.fori_loop(0, rpt // zr, zb, 0)
        plsc.subcore_barrier()

        base_e = wid * epw

        def body(b, _):
            pltpu.sync_copy(dst_hbm.at[pl.ds(base_e + b * blk, blk)], dst_v)
            pltpu.sync_copy(ones_v, acc.at[dst_v], add=True)
            return 0

        lax.fori_loop(0, nblk, body, 0)
        plsc.subcore_barrier()

        pltpu.sync_copy(acc.at[pl.ds(row0, rpt)], obuf)
        pltpu.sync_copy(obuf, out_hbm.at[c].at[pl.ds(row0, rpt)])

    return k(dst)


@functools.partial(jax.jit, static_argnames=("n", "e", "d"))
def _sc_scatter(y, src, dst, *, n, e, d):
    nw = NC * NS
    epw = e // nw
    blk = 80
    nblk = epw // blk
    rpt = n // NS
    zr = 25
    ob = 125

    mesh = plsc.VectorSubcoreMesh(
        core_axis_name="c", subcore_axis_name="s",
        num_cores=NC, num_subcores=NS)

    @functools.partial(
        pl.kernel,
        out_type=jax.ShapeDtypeStruct((NC, n, d), jnp.float32),
        mesh=mesh,
        scratch_types=[
            pltpu.VMEM((blk,), jnp.int32),
            pltpu.VMEM((blk,), jnp.int32),
            pltpu.VMEM((blk, d), jnp.float32),
            pltpu.VMEM((zr, d), jnp.float32),
            pltpu.VMEM((ob, d), jnp.float32),
            pltpu.VMEM_SHARED((n, d), jnp.float32),
            pltpu.SemaphoreType.DMA,
        ],
    )
    def k(y_hbm, src_hbm, dst_hbm, out_hbm,
          src_v, dst_v, rows_v, zbuf, obuf, acc, sem):
        c = lax.axis_index("c")
        s = lax.axis_index("s")
        wid = s * NC + c
        zero = jnp.zeros((LANES,), jnp.float32)
        for r in range(zr):
            for kk in range(d // LANES):
                zbuf[r, pl.ds(kk * LANES, LANES)] = zero
        row0 = s * rpt

        def zb(i, _):
            pltpu.sync_copy(zbuf, acc.at[pl.ds(row0 + i * zr, zr)])
            return 0

        lax.fori_loop(0, rpt // zr, zb, 0)
        plsc.subcore_barrier()

        base_e = wid * epw

        def body(b, _):
            e0 = base_e + b * blk
            pltpu.sync_copy(src_hbm.at[pl.ds(e0, blk)], src_v)
            pltpu.sync_copy(dst_hbm.at[pl.ds(e0, blk)], dst_v)
            pltpu.async_copy(y_hbm.at[src_v], rows_v, sem).wait()
            pltpu.sync_copy(rows_v, acc.at[dst_v], add=True)
            return 0

        lax.fori_loop(0, nblk, body, 0)
        plsc.subcore_barrier()

        def wb(i, _):
            r0 = row0 + i * ob
            pltpu.sync_copy(acc.at[pl.ds(r0, ob)], obuf)
            pltpu.sync_copy(obuf, out_hbm.at[c].at[pl.ds(r0, ob)])
            return 0

        lax.fori_loop(0, rpt // ob, wb, 0)

    return k(y, src, dst)


_RB = 1000


def _dinv_block(cnt_ref):
    deg = cnt_ref[0, :, 0:1] + cnt_ref[1, :, 0:1] + 1.0
    return lax.rsqrt(deg)


def _dense1_body(cnt_ref, x_ref, w_ref, y_ref, xw_ref):
    dinv = _dinv_block(cnt_ref)
    xw = jnp.dot(x_ref[...], w_ref[...], preferred_element_type=jnp.float32)
    xw_ref[...] = xw
    y_ref[...] = xw * dinv


def _dense2_body(sp_ref, xw1_ref, cnt_ref, b_ref, w_ref, y_ref, xw_ref):
    dinv = _dinv_block(cnt_ref)
    pre = dinv * (sp_ref[0] + sp_ref[1]) \
        + (dinv * dinv) * xw1_ref[...] + b_ref[...]
    h = jnp.maximum(pre, 0.0)
    xw = jnp.dot(h, w_ref[...], preferred_element_type=jnp.float32)
    xw_ref[...] = xw
    y_ref[...] = xw * dinv


def _dense3_body(sp_ref, xw2_ref, cnt_ref, b_ref, out_ref):
    dinv = _dinv_block(cnt_ref)
    o = dinv * (sp_ref[0] + sp_ref[1]) \
        + (dinv * dinv) * xw2_ref[...] + b_ref[...]
    m = jnp.max(o, axis=1, keepdims=True)
    lse = jnp.log(jnp.sum(jnp.exp(o - m), axis=1, keepdims=True))
    out_ref[...] = o - m - lse


def _dense1(cnt, x, w1, *, n, d):
    grid = (n // _RB,)
    return pl.pallas_call(
        _dense1_body,
        grid=grid,
        in_specs=[
            pl.BlockSpec((NC, _RB, LANES), lambda i: (0, i, 0)),
            pl.BlockSpec((_RB, d), lambda i: (i, 0)),
            pl.BlockSpec((d, d), lambda i: (0, 0)),
        ],
        out_specs=[
            pl.BlockSpec((_RB, d), lambda i: (i, 0)),
            pl.BlockSpec((_RB, d), lambda i: (i, 0)),
        ],
        out_shape=[
            jax.ShapeDtypeStruct((n, d), jnp.float32),
            jax.ShapeDtypeStruct((n, d), jnp.float32),
        ],
    )(cnt, x, w1)


def _dense2(sp, xw1, cnt, b1, w2, *, n, d):
    grid = (n // _RB,)
    return pl.pallas_call(
        _dense2_body,
        grid=grid,
        in_specs=[
            pl.BlockSpec((NC, _RB, d), lambda i: (0, i, 0)),
            pl.BlockSpec((_RB, d), lambda i: (i, 0)),
            pl.BlockSpec((NC, _RB, LANES), lambda i: (0, i, 0)),
            pl.BlockSpec((1, d), lambda i: (0, 0)),
            pl.BlockSpec((d, d), lambda i: (0, 0)),
        ],
        out_specs=[
            pl.BlockSpec((_RB, d), lambda i: (i, 0)),
            pl.BlockSpec((_RB, d), lambda i: (i, 0)),
        ],
        out_shape=[
            jax.ShapeDtypeStruct((n, d), jnp.float32),
            jax.ShapeDtypeStruct((n, d), jnp.float32),
        ],
    )(sp, xw1, cnt, b1, w2)


def _dense3(sp, xw2, cnt, b2, *, n, d):
    grid = (n // _RB,)
    return pl.pallas_call(
        _dense3_body,
        grid=grid,
        in_specs=[
            pl.BlockSpec((NC, _RB, d), lambda i: (0, i, 0)),
            pl.BlockSpec((_RB, d), lambda i: (i, 0)),
            pl.BlockSpec((NC, _RB, LANES), lambda i: (0, i, 0)),
            pl.BlockSpec((1, d), lambda i: (0, 0)),
        ],
        out_specs=pl.BlockSpec((_RB, d), lambda i: (i, 0)),
        out_shape=jax.ShapeDtypeStruct((n, d), jnp.float32),
    )(sp, xw2, cnt, b2)


def kernel(x, edge_index, W1, b1, W2, b2):
    n, d = x.shape
    e = edge_index.shape[1]
    src = edge_index[0]
    dst = edge_index[1]

    cnt = _sc_count(dst, n=n, e=e)
    y1, xw1 = _dense1(cnt, x, W1, n=n, d=d)
    s1 = _sc_scatter(y1, src, dst, n=n, e=e, d=d)
    y2, xw2 = _dense2(s1, xw1, cnt, b1.reshape(1, d), W2, n=n, d=d)
    s2 = _sc_scatter(y2, src, dst, n=n, e=e, d=d)
    return _dense3(s2, xw2, cnt, b2.reshape(1, d), n=n, d=d)


# speedup vs baseline: 12.0175x; 12.0175x over previous
import functools

import jax
import jax.numpy as jnp
from jax import lax
from jax.experimental import pallas as pl
from jax.experimental.pallas import tpu as pltpu
from jax.experimental.pallas import tpu_sc as plsc

NC = 2
NS = 16
LANES = 16


def _pad_rows(n):
    rpt = -(-n // NS)
    rpt = -(-rpt // 128) * 128
    return rpt, rpt * NS


@functools.partial(jax.jit, static_argnames=("n", "e", "d"))
def _sc_count(dst, *, n, e, d):
    nw = NC * NS
    epw = e // nw
    blk = 80
    nblk = epw // blk
    rpt, n_pad = _pad_rows(n)
    zr = 32
    ob = 128

    mesh = plsc.VectorSubcoreMesh(
        core_axis_name="c", subcore_axis_name="s",
        num_cores=NC, num_subcores=NS)

    @functools.partial(
        pl.kernel,
        out_type=jax.ShapeDtypeStruct((NC, n_pad, d), jnp.float32),
        mesh=mesh,
        scratch_types=[
            pltpu.VMEM((blk,), jnp.int32),
            pltpu.VMEM((blk, d), jnp.float32),
            pltpu.VMEM((zr, d), jnp.float32),
            pltpu.VMEM((ob, d), jnp.float32),
            pltpu.VMEM_SHARED((n_pad, d), jnp.float32),
        ],
    )
    def k(dst_hbm, out_hbm, dst_v, ones_v, zbuf, obuf, acc):
        c = lax.axis_index("c")
        s = lax.axis_index("s")
        wid = s * NC + c
        one = jnp.ones((LANES,), jnp.float32)
        zero = jnp.zeros((LANES,), jnp.float32)
        for r in range(blk):
            for kk in range(d // LANES):
                ones_v[r, pl.ds(kk * LANES, LANES)] = one
        for r in range(zr):
            for kk in range(d // LANES):
                zbuf[r, pl.ds(kk * LANES, LANES)] = zero
        row0 = s * rpt

        def zb(i, _):
            pltpu.sync_copy(zbuf, acc.at[pl.ds(row0 + i * zr, zr)])
            return 0

        lax.fori_loop(0, rpt // zr, zb, 0)
        plsc.subcore_barrier()

        base_e = wid * epw

        def body(b, _):
            pltpu.sync_copy(dst_hbm.at[pl.ds(base_e + b * blk, blk)], dst_v)
            pltpu.sync_copy(ones_v, acc.at[dst_v], add=True)
            return 0

        lax.fori_loop(0, nblk, body, 0)
        plsc.subcore_barrier()

        def wb(i, _):
            r0 = row0 + i * ob
            pltpu.sync_copy(acc.at[pl.ds(r0, ob)], obuf)
            pltpu.sync_copy(obuf, out_hbm.at[c].at[pl.ds(r0, ob)])
            return 0

        lax.fori_loop(0, rpt // ob, wb, 0)

    return k(dst)[:, :n, :]


@functools.partial(jax.jit, static_argnames=("n", "e", "d"))
def _sc_scatter(y, src, dst, *, n, e, d):
    nw = NC * NS
    epw = e // nw
    blk = 80
    nblk = epw // blk
    rpt, n_pad = _pad_rows(n)
    zr = 32
    ob = 128

    mesh = plsc.VectorSubcoreMesh(
        core_axis_name="c", subcore_axis_name="s",
        num_cores=NC, num_subcores=NS)

    @functools.partial(
        pl.kernel,
        out_type=jax.ShapeDtypeStruct((NC, n_pad, d), jnp.float32),
        mesh=mesh,
        scratch_types=[
            pltpu.VMEM((blk,), jnp.int32),
            pltpu.VMEM((blk,), jnp.int32),
            pltpu.VMEM((blk, d), jnp.float32),
            pltpu.VMEM((zr, d), jnp.float32),
            pltpu.VMEM((ob, d), jnp.float32),
            pltpu.VMEM_SHARED((n_pad, d), jnp.float32),
            pltpu.SemaphoreType.DMA,
        ],
    )
    def k(y_hbm, src_hbm, dst_hbm, out_hbm,
          src_v, dst_v, rows_v, zbuf, obuf, acc, sem):
        c = lax.axis_index("c")
        s = lax.axis_index("s")
        wid = s * NC + c
        zero = jnp.zeros((LANES,), jnp.float32)
        for r in range(zr):
            for kk in range(d // LANES):
                zbuf[r, pl.ds(kk * LANES, LANES)] = zero
        row0 = s * rpt

        def zb(i, _):
            pltpu.sync_copy(zbuf, acc.at[pl.ds(row0 + i * zr, zr)])
            return 0

        lax.fori_loop(0, rpt // zr, zb, 0)
        plsc.subcore_barrier()

        base_e = wid * epw

        def body(b, _):
            e0 = base_e + b * blk
            pltpu.sync_copy(src_hbm.at[pl.ds(e0, blk)], src_v)
            pltpu.sync_copy(dst_hbm.at[pl.ds(e0, blk)], dst_v)
            pltpu.async_copy(y_hbm.at[src_v], rows_v, sem).wait()
            pltpu.sync_copy(rows_v, acc.at[dst_v], add=True)
            return 0

        lax.fori_loop(0, nblk, body, 0)
        plsc.subcore_barrier()

        def wb(i, _):
            r0 = row0 + i * ob
            pltpu.sync_copy(acc.at[pl.ds(r0, ob)], obuf)
            pltpu.sync_copy(obuf, out_hbm.at[c].at[pl.ds(r0, ob)])
            return 0

        lax.fori_loop(0, rpt // ob, wb, 0)

    return k(y, src, dst)[:, :n, :]


_RB = 1000


def _dinv_block(cnt_ref):
    deg = cnt_ref[0, :, 0:1] + cnt_ref[1, :, 0:1] + 1.0
    return lax.rsqrt(deg)


def _dense1_body(cnt_ref, x_ref, w_ref, y_ref, xw_ref):
    dinv = _dinv_block(cnt_ref)
    xw = jnp.dot(x_ref[...], w_ref[...], preferred_element_type=jnp.float32)
    xw_ref[...] = xw
    y_ref[...] = xw * dinv


def _dense2_body(sp_ref, xw1_ref, cnt_ref, b_ref, w_ref, y_ref, xw_ref):
    dinv = _dinv_block(cnt_ref)
    pre = dinv * (sp_ref[0] + sp_ref[1]) \
        + (dinv * dinv) * xw1_ref[...] + b_ref[...]
    h = jnp.maximum(pre, 0.0)
    xw = jnp.dot(h, w_ref[...], preferred_element_type=jnp.float32)
    xw_ref[...] = xw
    y_ref[...] = xw * dinv


def _dense3_body(sp_ref, xw2_ref, cnt_ref, b_ref, out_ref):
    dinv = _dinv_block(cnt_ref)
    o = dinv * (sp_ref[0] + sp_ref[1]) \
        + (dinv * dinv) * xw2_ref[...] + b_ref[...]
    m = jnp.max(o, axis=1, keepdims=True)
    lse = jnp.log(jnp.sum(jnp.exp(o - m), axis=1, keepdims=True))
    out_ref[...] = o - m - lse


def _dense1(cnt, x, w1, *, n, d):
    grid = (n // _RB,)
    return pl.pallas_call(
        _dense1_body,
        grid=grid,
        in_specs=[
            pl.BlockSpec((NC, _RB, d), lambda i: (0, i, 0)),
            pl.BlockSpec((_RB, d), lambda i: (i, 0)),
            pl.BlockSpec((d, d), lambda i: (0, 0)),
        ],
        out_specs=[
            pl.BlockSpec((_RB, d), lambda i: (i, 0)),
            pl.BlockSpec((_RB, d), lambda i: (i, 0)),
        ],
        out_shape=[
            jax.ShapeDtypeStruct((n, d), jnp.float32),
            jax.ShapeDtypeStruct((n, d), jnp.float32),
        ],
    )(cnt, x, w1)


def _dense2(sp, xw1, cnt, b1, w2, *, n, d):
    grid = (n // _RB,)
    return pl.pallas_call(
        _dense2_body,
        grid=grid,
        in_specs=[
            pl.BlockSpec((NC, _RB, d), lambda i: (0, i, 0)),
            pl.BlockSpec((_RB, d), lambda i: (i, 0)),
            pl.BlockSpec((NC, _RB, d), lambda i: (0, i, 0)),
            pl.BlockSpec((1, d), lambda i: (0, 0)),
            pl.BlockSpec((d, d), lambda i: (0, 0)),
        ],
        out_specs=[
            pl.BlockSpec((_RB, d), lambda i: (i, 0)),
            pl.BlockSpec((_RB, d), lambda i: (i, 0)),
        ],
        out_shape=[
            jax.ShapeDtypeStruct((n, d), jnp.float32),
            jax.ShapeDtypeStruct((n, d), jnp.float32),
        ],
    )(sp, xw1, cnt, b1, w2)


def _dense3(sp, xw2, cnt, b2, *, n, d):
    grid = (n // _RB,)
    return pl.pallas_call(
        _dense3_body,
        grid=grid,
        in_specs=[
            pl.BlockSpec((NC, _RB, d), lambda i: (0, i, 0)),
            pl.BlockSpec((_RB, d), lambda i: (i, 0)),
            pl.BlockSpec((NC, _RB, d), lambda i: (0, i, 0)),
            pl.BlockSpec((1, d), lambda i: (0, 0)),
        ],
        out_specs=pl.BlockSpec((_RB, d), lambda i: (i, 0)),
        out_shape=jax.ShapeDtypeStruct((n, d), jnp.float32),
    )(sp, xw2, cnt, b2)


def kernel(x, edge_index, W1, b1, W2, b2):
    n, d = x.shape
    e = edge_index.shape[1]
    src = edge_index[0]
    dst = edge_index[1]

    cnt = _sc_count(dst, n=n, e=e, d=d)
    y1, xw1 = _dense1(cnt, x, W1, n=n, d=d)
    s1 = _sc_scatter(y1, src, dst, n=n, e=e, d=d)
    y2, xw2 = _dense2(s1, xw1, cnt, b1.reshape(1, d), W2, n=n, d=d)
    s2 = _sc_scatter(y2, src, dst, n=n, e=e, d=d)
    return _dense3(s2, xw2, cnt, b2.reshape(1, d), n=n, d=d)


# speedup vs baseline: 19.5097x; 1.6234x over previous
import functools

import jax
import jax.numpy as jnp
from jax import lax
from jax.experimental import pallas as pl
from jax.experimental.pallas import tpu as pltpu
from jax.experimental.pallas import tpu_sc as plsc

NC = 2
NS = 16
LANES = 16


def _pad_rows(n):
    rpt = -(-n // NS)
    rpt = -(-rpt // 128) * 128
    return rpt, rpt * NS


@functools.partial(jax.jit, static_argnames=("n", "e", "d"))
def _sc_count(dst, *, n, e, d):
    nw = NC * NS
    epw = e // nw
    blk = 80
    nblk = epw // blk
    rpt, n_pad = _pad_rows(n)
    zr = 32
    ob = 128

    mesh = plsc.VectorSubcoreMesh(
        core_axis_name="c", subcore_axis_name="s",
        num_cores=NC, num_subcores=NS)

    @functools.partial(
        pl.kernel,
        out_type=jax.ShapeDtypeStruct((NC, n_pad, d), jnp.float32),
        mesh=mesh,
        scratch_types=[
            pltpu.VMEM((blk,), jnp.int32),
            pltpu.VMEM((blk, d), jnp.float32),
            pltpu.VMEM((zr, d), jnp.float32),
            pltpu.VMEM((ob, d), jnp.float32),
            pltpu.VMEM_SHARED((n_pad, d), jnp.float32),
        ],
    )
    def k(dst_hbm, out_hbm, dst_v, ones_v, zbuf, obuf, acc):
        c = lax.axis_index("c")
        s = lax.axis_index("s")
        wid = s * NC + c
        one = jnp.ones((LANES,), jnp.float32)
        zero = jnp.zeros((LANES,), jnp.float32)
        for r in range(blk):
            for kk in range(d // LANES):
                ones_v[r, pl.ds(kk * LANES, LANES)] = one
        for r in range(zr):
            for kk in range(d // LANES):
                zbuf[r, pl.ds(kk * LANES, LANES)] = zero
        row0 = s * rpt

        def zb(i, _):
            pltpu.sync_copy(zbuf, acc.at[pl.ds(row0 + i * zr, zr)])
            return 0

        lax.fori_loop(0, rpt // zr, zb, 0)
        plsc.subcore_barrier()

        base_e = wid * epw

        def body(b, _):
            pltpu.sync_copy(dst_hbm.at[pl.ds(base_e + b * blk, blk)], dst_v)
            pltpu.sync_copy(ones_v, acc.at[dst_v], add=True)
            return 0

        lax.fori_loop(0, nblk, body, 0)
        plsc.subcore_barrier()

        def wb(i, _):
            r0 = row0 + i * ob
            pltpu.sync_copy(acc.at[pl.ds(r0, ob)], obuf)
            pltpu.sync_copy(obuf, out_hbm.at[c].at[pl.ds(r0, ob)])
            return 0

        lax.fori_loop(0, rpt // ob, wb, 0)

    return k(dst)[:, :n, :]


@functools.partial(jax.jit, static_argnames=("n", "e", "d"))
def _sc_scatter(y, src, dst, *, n, e, d):
    nw = NC * NS
    epw = e // nw
    blk = 128
    nblk = epw // blk
    tail = epw - nblk * blk
    rpt, n_pad = _pad_rows(n)
    zr = 16
    ob = 64

    mesh = plsc.VectorSubcoreMesh(
        core_axis_name="c", subcore_axis_name="s",
        num_cores=NC, num_subcores=NS)

    @functools.partial(
        pl.kernel,
        out_type=jax.ShapeDtypeStruct((NC, n_pad, d), jnp.float32),
        mesh=mesh,
        scratch_types=[
            pltpu.VMEM((blk,), jnp.int32),
            pltpu.VMEM((blk,), jnp.int32),
            pltpu.VMEM((blk,), jnp.int32),
            pltpu.VMEM((blk,), jnp.int32),
            pltpu.VMEM((blk, d), jnp.float32),
            pltpu.VMEM((blk, d), jnp.float32),
            pltpu.VMEM((max(tail, 8),), jnp.int32),
            pltpu.VMEM((max(tail, 8),), jnp.int32),
            pltpu.VMEM((max(tail, 1), d), jnp.float32),
            pltpu.VMEM((zr, d), jnp.float32),
            pltpu.VMEM((ob, d), jnp.float32),
            pltpu.VMEM_SHARED((n_pad, d), jnp.float32),
            pltpu.SemaphoreType.DMA,
            pltpu.SemaphoreType.DMA,
        ],
    )
    def k(y_hbm, src_hbm, dst_hbm, out_hbm,
          src_a, dst_a, src_b, dst_b, rows_a, rows_b,
          src_t, dst_t, rows_t, zbuf, obuf, acc, sem_a, sem_b):
        c = lax.axis_index("c")
        s = lax.axis_index("s")
        wid = s * NC + c
        zero = jnp.zeros((LANES,), jnp.float32)
        for r in range(zr):
            for kk in range(d // LANES):
                zbuf[r, pl.ds(kk * LANES, LANES)] = zero
        row0 = s * rpt

        def zb(i, _):
            pltpu.sync_copy(zbuf, acc.at[pl.ds(row0 + i * zr, zr)])
            return 0

        lax.fori_loop(0, rpt // zr, zb, 0)
        plsc.subcore_barrier()

        base_e = wid * epw

        pltpu.sync_copy(src_hbm.at[pl.ds(base_e, blk)], src_a)
        pltpu.sync_copy(dst_hbm.at[pl.ds(base_e, blk)], dst_a)
        pltpu.async_copy(y_hbm.at[src_a], rows_a, sem_a)

        def body(i, _):
            nxt = i + 1

            def stage(src_c, dst_c, rows_c, sem_c,
                      src_n, dst_n, rows_n, sem_n):
                @pl.when(nxt < nblk)
                def _():
                    e0n = base_e + nxt * blk
                    pltpu.sync_copy(src_hbm.at[pl.ds(e0n, blk)], src_n)
                    pltpu.sync_copy(dst_hbm.at[pl.ds(e0n, blk)], dst_n)
                    pltpu.async_copy(y_hbm.at[src_n], rows_n, sem_n)

                pltpu.make_async_copy(y_hbm.at[src_c], rows_c, sem_c).wait()
                pltpu.sync_copy(rows_c, acc.at[dst_c], add=True)

            @pl.when(i % 2 == 0)
            def _():
                stage(src_a, dst_a, rows_a, sem_a,
                      src_b, dst_b, rows_b, sem_b)

            @pl.when(i % 2 == 1)
            def _():
                stage(src_b, dst_b, rows_b, sem_b,
                      src_a, dst_a, rows_a, sem_a)

            return 0

        lax.fori_loop(0, nblk, body, 0)

        if tail:
            e0t = base_e + nblk * blk
            pltpu.sync_copy(src_hbm.at[pl.ds(e0t, tail)],
                            src_t.at[pl.ds(0, tail)])
            pltpu.sync_copy(dst_hbm.at[pl.ds(e0t, tail)],
                            dst_t.at[pl.ds(0, tail)])
            pltpu.async_copy(y_hbm.at[src_t], rows_t, sem_a).wait()
            pltpu.sync_copy(rows_t, acc.at[dst_t], add=True)

        plsc.subcore_barrier()

        def wb(i, _):
            r0 = row0 + i * ob
            pltpu.sync_copy(acc.at[pl.ds(r0, ob)], obuf)
            pltpu.sync_copy(obuf, out_hbm.at[c].at[pl.ds(r0, ob)])
            return 0

        lax.fori_loop(0, rpt // ob, wb, 0)

    return k(y, src, dst)[:, :n, :]


_RB = 1000


def _dinv_block(cnt_ref):
    deg = cnt_ref[0, :, 0:1] + cnt_ref[1, :, 0:1] + 1.0
    return lax.rsqrt(deg)


def _dense1_body(cnt_ref, x_ref, w_ref, y_ref, xw_ref):
    dinv = _dinv_block(cnt_ref)
    xw = jnp.dot(x_ref[...], w_ref[...], preferred_element_type=jnp.float32)
    xw_ref[...] = xw
    y_ref[...] = xw * dinv


def _dense2_body(sp_ref, xw1_ref, cnt_ref, b_ref, w_ref, y_ref, xw_ref):
    dinv = _dinv_block(cnt_ref)
    pre = dinv * (sp_ref[0] + sp_ref[1]) \
        + (dinv * dinv) * xw1_ref[...] + b_ref[...]
    h = jnp.maximum(pre, 0.0)
    xw = jnp.dot(h, w_ref[...], preferred_element_type=jnp.float32)
    xw_ref[...] = xw
    y_ref[...] = xw * dinv


def _dense3_body(sp_ref, xw2_ref, cnt_ref, b_ref, out_ref):
    dinv = _dinv_block(cnt_ref)
    o = dinv * (sp_ref[0] + sp_ref[1]) \
        + (dinv * dinv) * xw2_ref[...] + b_ref[...]
    m = jnp.max(o, axis=1, keepdims=True)
    lse = jnp.log(jnp.sum(jnp.exp(o - m), axis=1, keepdims=True))
    out_ref[...] = o - m - lse


def _dense1(cnt, x, w1, *, n, d):
    grid = (n // _RB,)
    return pl.pallas_call(
        _dense1_body,
        grid=grid,
        in_specs=[
            pl.BlockSpec((NC, _RB, d), lambda i: (0, i, 0)),
            pl.BlockSpec((_RB, d), lambda i: (i, 0)),
            pl.BlockSpec((d, d), lambda i: (0, 0)),
        ],
        out_specs=[
            pl.BlockSpec((_RB, d), lambda i: (i, 0)),
            pl.BlockSpec((_RB, d), lambda i: (i, 0)),
        ],
        out_shape=[
            jax.ShapeDtypeStruct((n, d), jnp.float32),
            jax.ShapeDtypeStruct((n, d), jnp.float32),
        ],
    )(cnt, x, w1)


def _dense2(sp, xw1, cnt, b1, w2, *, n, d):
    grid = (n // _RB,)
    return pl.pallas_call(
        _dense2_body,
        grid=grid,
        in_specs=[
            pl.BlockSpec((NC, _RB, d), lambda i: (0, i, 0)),
            pl.BlockSpec((_RB, d), lambda i: (i, 0)),
            pl.BlockSpec((NC, _RB, d), lambda i: (0, i, 0)),
            pl.BlockSpec((1, d), lambda i: (0, 0)),
            pl.BlockSpec((d, d), lambda i: (0, 0)),
        ],
        out_specs=[
            pl.BlockSpec((_RB, d), lambda i: (i, 0)),
            pl.BlockSpec((_RB, d), lambda i: (i, 0)),
        ],
        out_shape=[
            jax.ShapeDtypeStruct((n, d), jnp.float32),
            jax.ShapeDtypeStruct((n, d), jnp.float32),
        ],
    )(sp, xw1, cnt, b1, w2)


def _dense3(sp, xw2, cnt, b2, *, n, d):
    grid = (n // _RB,)
    return pl.pallas_call(
        _dense3_body,
        grid=grid,
        in_specs=[
            pl.BlockSpec((NC, _RB, d), lambda i: (0, i, 0)),
            pl.BlockSpec((_RB, d), lambda i: (i, 0)),
            pl.BlockSpec((NC, _RB, d), lambda i: (0, i, 0)),
            pl.BlockSpec((1, d), lambda i: (0, 0)),
        ],
        out_specs=pl.BlockSpec((_RB, d), lambda i: (i, 0)),
        out_shape=jax.ShapeDtypeStruct((n, d), jnp.float32),
    )(sp, xw2, cnt, b2)


def kernel(x, edge_index, W1, b1, W2, b2):
    n, d = x.shape
    e = edge_index.shape[1]
    src = edge_index[0]
    dst = edge_index[1]

    cnt = _sc_count(dst, n=n, e=e, d=d)
    y1, xw1 = _dense1(cnt, x, W1, n=n, d=d)
    s1 = _sc_scatter(y1, src, dst, n=n, e=e, d=d)
    y2, xw2 = _dense2(s1, xw1, cnt, b1.reshape(1, d), W2, n=n, d=d)
    s2 = _sc_scatter(y2, src, dst, n=n, e=e, d=d)
    return _dense3(s2, xw2, cnt, b2.reshape(1, d), n=n, d=d)


# speedup vs baseline: 24.3972x; 1.2505x over previous
import functools

import jax
import jax.numpy as jnp
from jax import lax
from jax.experimental import pallas as pl
from jax.experimental.pallas import tpu as pltpu
from jax.experimental.pallas import tpu_sc as plsc

NC = 2
NS = 16
LANES = 16


def _pad_rows(n):
    rpt = -(-n // NS)
    rpt = -(-rpt // 128) * 128
    return rpt, rpt * NS


@functools.partial(jax.jit, static_argnames=("n", "e", "d"))
def _sc_count(dst, *, n, e, d):
    nw = NC * NS
    epw = e // nw
    blk = 128
    nblk = epw // blk
    tail = epw - nblk * blk
    rpt, n_pad = _pad_rows(n)
    zr = 32
    ob = 128

    mesh = plsc.VectorSubcoreMesh(
        core_axis_name="c", subcore_axis_name="s",
        num_cores=NC, num_subcores=NS)

    @functools.partial(
        pl.kernel,
        out_type=jax.ShapeDtypeStruct((NC, n_pad, d), jnp.float32),
        mesh=mesh,
        scratch_types=[
            pltpu.VMEM((blk,), jnp.int32),
            pltpu.VMEM((blk,), jnp.int32),
            pltpu.VMEM((max(tail, 8),), jnp.int32),
            pltpu.VMEM((blk, d), jnp.float32),
            pltpu.VMEM((zr, d), jnp.float32),
            pltpu.VMEM((ob, d), jnp.float32),
            pltpu.VMEM_SHARED((n_pad, d), jnp.float32),
            pltpu.SemaphoreType.DMA,
            pltpu.SemaphoreType.DMA,
        ],
    )
    def k(dst_hbm, out_hbm, dst_a, dst_b, dst_t, ones_v, zbuf, obuf, acc,
          isem_a, isem_b):
        c = lax.axis_index("c")
        s = lax.axis_index("s")
        wid = s * NC + c
        one = jnp.ones((LANES,), jnp.float32)
        zero = jnp.zeros((LANES,), jnp.float32)
        for r in range(blk):
            for kk in range(d // LANES):
                ones_v[r, pl.ds(kk * LANES, LANES)] = one
        for r in range(zr):
            for kk in range(d // LANES):
                zbuf[r, pl.ds(kk * LANES, LANES)] = zero
        row0 = s * rpt

        def zb(i, _):
            pltpu.sync_copy(zbuf, acc.at[pl.ds(row0 + i * zr, zr)])
            return 0

        lax.fori_loop(0, rpt // zr, zb, 0)
        plsc.subcore_barrier()

        base_e = wid * epw

        def idx_load(b, dst_n, sem_n):
            pltpu.async_copy(dst_hbm.at[pl.ds(base_e + b * blk, blk)],
                             dst_n, sem_n)

        def idx_wait(dst_n, sem_n):
            pltpu.make_async_copy(dst_hbm.at[pl.ds(base_e, blk)],
                                  dst_n, sem_n).wait()

        idx_load(0, dst_a, isem_a)

        def body(i, _):
            def stage(dst_c, isem_c, dst_n, isem_n):
                @pl.when(i + 1 < nblk)
                def _():
                    idx_load(i + 1, dst_n, isem_n)

                idx_wait(dst_c, isem_c)
                pltpu.sync_copy(ones_v, acc.at[dst_c], add=True)

            @pl.when(i % 2 == 0)
            def _():
                stage(dst_a, isem_a, dst_b, isem_b)

            @pl.when(i % 2 == 1)
            def _():
                stage(dst_b, isem_b, dst_a, isem_a)

            return 0

        lax.fori_loop(0, nblk, body, 0)

        if tail:
            e0t = base_e + nblk * blk
            pltpu.sync_copy(dst_hbm.at[pl.ds(e0t, tail)],
                            dst_t.at[pl.ds(0, tail)])
            pltpu.sync_copy(ones_v.at[pl.ds(0, tail)],
                            acc.at[dst_t], add=True)

        plsc.subcore_barrier()

        def wb(i, _):
            r0 = row0 + i * ob
            pltpu.sync_copy(acc.at[pl.ds(r0, ob)], obuf)
            pltpu.sync_copy(obuf, out_hbm.at[c].at[pl.ds(r0, ob)])
            return 0

        lax.fori_loop(0, rpt // ob, wb, 0)

    return k(dst)[:, :n, :]


@functools.partial(jax.jit, static_argnames=("n", "e", "d"))
def _sc_scatter(y, src, dst, *, n, e, d):
    nw = NC * NS
    epw = e // nw
    blk = 128
    nblk = epw // blk
    tail = epw - nblk * blk
    rpt, n_pad = _pad_rows(n)
    zr = 16
    ob = 64

    mesh = plsc.VectorSubcoreMesh(
        core_axis_name="c", subcore_axis_name="s",
        num_cores=NC, num_subcores=NS)

    @functools.partial(
        pl.kernel,
        out_type=jax.ShapeDtypeStruct((NC, n_pad, d), jnp.float32),
        mesh=mesh,
        scratch_types=[
            pltpu.VMEM((blk,), jnp.int32),
            pltpu.VMEM((blk,), jnp.int32),
            pltpu.VMEM((blk,), jnp.int32),
            pltpu.VMEM((blk,), jnp.int32),
            pltpu.VMEM((blk, d), jnp.float32),
            pltpu.VMEM((blk, d), jnp.float32),
            pltpu.VMEM((max(tail, 8),), jnp.int32),
            pltpu.VMEM((max(tail, 8),), jnp.int32),
            pltpu.VMEM((max(tail, 1), d), jnp.float32),
            pltpu.VMEM((zr, d), jnp.float32),
            pltpu.VMEM((ob, d), jnp.float32),
            pltpu.VMEM_SHARED((n_pad, d), jnp.float32),
            pltpu.SemaphoreType.DMA,
            pltpu.SemaphoreType.DMA,
            pltpu.SemaphoreType.DMA,
            pltpu.SemaphoreType.DMA,
        ],
    )
    def k(y_hbm, src_hbm, dst_hbm, out_hbm,
          src_a, dst_a, src_b, dst_b, rows_a, rows_b,
          src_t, dst_t, rows_t, zbuf, obuf, acc,
          sem_a, sem_b, isem_a, isem_b):
        c = lax.axis_index("c")
        s = lax.axis_index("s")
        wid = s * NC + c
        zero = jnp.zeros((LANES,), jnp.float32)
        for r in range(zr):
            for kk in range(d // LANES):
                zbuf[r, pl.ds(kk * LANES, LANES)] = zero
        row0 = s * rpt

        def zb(i, _):
            pltpu.sync_copy(zbuf, acc.at[pl.ds(row0 + i * zr, zr)])
            return 0

        lax.fori_loop(0, rpt // zr, zb, 0)
        plsc.subcore_barrier()

        base_e = wid * epw

        def idx_load(b, src_n, dst_n, sem_n):
            e0n = base_e + b * blk
            pltpu.async_copy(src_hbm.at[pl.ds(e0n, blk)], src_n, sem_n)
            pltpu.async_copy(dst_hbm.at[pl.ds(e0n, blk)], dst_n, sem_n)

        def idx_wait(src_n, dst_n, sem_n):
            pltpu.make_async_copy(src_hbm.at[pl.ds(base_e, blk)],
                                  src_n, sem_n).wait()
            pltpu.make_async_copy(dst_hbm.at[pl.ds(base_e, blk)],
                                  dst_n, sem_n).wait()

        idx_load(0, src_a, dst_a, isem_a)
        idx_wait(src_a, dst_a, isem_a)
        pltpu.async_copy(y_hbm.at[src_a], rows_a, sem_a)
        if nblk > 1:
            idx_load(1, src_b, dst_b, isem_b)

        def body(i, _):
            def stage(src_c, dst_c, rows_c, sem_c, isem_c,
                      src_n, dst_n, rows_n, sem_n, isem_n):
                @pl.when(i + 1 < nblk)
                def _():
                    idx_wait(src_n, dst_n, isem_n)
                    pltpu.async_copy(y_hbm.at[src_n], rows_n, sem_n)

                pltpu.make_async_copy(y_hbm.at[src_c], rows_c, sem_c).wait()
                pltpu.sync_copy(rows_c, acc.at[dst_c], add=True)

                @pl.when(i + 2 < nblk)
                def _():
                    idx_load(i + 2, src_c, dst_c, isem_c)

            @pl.when(i % 2 == 0)
            def _():
                stage(src_a, dst_a, rows_a, sem_a, isem_a,
                      src_b, dst_b, rows_b, sem_b, isem_b)

            @pl.when(i % 2 == 1)
            def _():
                stage(src_b, dst_b, rows_b, sem_b, isem_b,
                      src_a, dst_a, rows_a, sem_a, isem_a)

            return 0

        lax.fori_loop(0, nblk, body, 0)

        if tail:
            e0t = base_e + nblk * blk
            pltpu.sync_copy(src_hbm.at[pl.ds(e0t, tail)],
                            src_t.at[pl.ds(0, tail)])
            pltpu.sync_copy(dst_hbm.at[pl.ds(e0t, tail)],
                            dst_t.at[pl.ds(0, tail)])
            pltpu.async_copy(y_hbm.at[src_t], rows_t, sem_a).wait()
            pltpu.sync_copy(rows_t, acc.at[dst_t], add=True)

        plsc.subcore_barrier()

        def wb(i, _):
            r0 = row0 + i * ob
            pltpu.sync_copy(acc.at[pl.ds(r0, ob)], obuf)
            pltpu.sync_copy(obuf, out_hbm.at[c].at[pl.ds(r0, ob)])
            return 0

        lax.fori_loop(0, rpt // ob, wb, 0)

    return k(y, src, dst)[:, :n, :]


_RB = 1000


def _dinv_block(cnt_ref):
    deg = cnt_ref[0, :, 0:1] + cnt_ref[1, :, 0:1] + 1.0
    return lax.rsqrt(deg)


def _dense1_body(cnt_ref, x_ref, w_ref, y_ref, xw_ref):
    dinv = _dinv_block(cnt_ref)
    xw = jnp.dot(x_ref[...], w_ref[...], preferred_element_type=jnp.float32)
    xw_ref[...] = xw
    y_ref[...] = xw * dinv


def _dense2_body(sp_ref, xw1_ref, cnt_ref, b_ref, w_ref, y_ref, xw_ref):
    dinv = _dinv_block(cnt_ref)
    pre = dinv * (sp_ref[0] + sp_ref[1]) \
        + (dinv * dinv) * xw1_ref[...] + b_ref[...]
    h = jnp.maximum(pre, 0.0)
    xw = jnp.dot(h, w_ref[...], preferred_element_type=jnp.float32)
    xw_ref[...] = xw
    y_ref[...] = xw * dinv


def _dense3_body(sp_ref, xw2_ref, cnt_ref, b_ref, out_ref):
    dinv = _dinv_block(cnt_ref)
    o = dinv * (sp_ref[0] + sp_ref[1]) \
        + (dinv * dinv) * xw2_ref[...] + b_ref[...]
    m = jnp.max(o, axis=1, keepdims=True)
    lse = jnp.log(jnp.sum(jnp.exp(o - m), axis=1, keepdims=True))
    out_ref[...] = o - m - lse


def _dense1(cnt, x, w1, *, n, d):
    grid = (n // _RB,)
    return pl.pallas_call(
        _dense1_body,
        grid=grid,
        in_specs=[
            pl.BlockSpec((NC, _RB, d), lambda i: (0, i, 0)),
            pl.BlockSpec((_RB, d), lambda i: (i, 0)),
            pl.BlockSpec((d, d), lambda i: (0, 0)),
        ],
        out_specs=[
            pl.BlockSpec((_RB, d), lambda i: (i, 0)),
            pl.BlockSpec((_RB, d), lambda i: (i, 0)),
        ],
        out_shape=[
            jax.ShapeDtypeStruct((n, d), jnp.float32),
            jax.ShapeDtypeStruct((n, d), jnp.float32),
        ],
    )(cnt, x, w1)


def _dense2(sp, xw1, cnt, b1, w2, *, n, d):
    grid = (n // _RB,)
    return pl.pallas_call(
        _dense2_body,
        grid=grid,
        in_specs=[
            pl.BlockSpec((NC, _RB, d), lambda i: (0, i, 0)),
            pl.BlockSpec((_RB, d), lambda i: (i, 0)),
            pl.BlockSpec((NC, _RB, d), lambda i: (0, i, 0)),
            pl.BlockSpec((1, d), lambda i: (0, 0)),
            pl.BlockSpec((d, d), lambda i: (0, 0)),
        ],
        out_specs=[
            pl.BlockSpec((_RB, d), lambda i: (i, 0)),
            pl.BlockSpec((_RB, d), lambda i: (i, 0)),
        ],
        out_shape=[
            jax.ShapeDtypeStruct((n, d), jnp.float32),
            jax.ShapeDtypeStruct((n, d), jnp.float32),
        ],
    )(sp, xw1, cnt, b1, w2)


def _dense3(sp, xw2, cnt, b2, *, n, d):
    grid = (n // _RB,)
    return pl.pallas_call(
        _dense3_body,
        grid=grid,
        in_specs=[
            pl.BlockSpec((NC, _RB, d), lambda i: (0, i, 0)),
            pl.BlockSpec((_RB, d), lambda i: (i, 0)),
            pl.BlockSpec((NC, _RB, d), lambda i: (0, i, 0)),
            pl.BlockSpec((1, d), lambda i: (0, 0)),
        ],
        out_specs=pl.BlockSpec((_RB, d), lambda i: (i, 0)),
        out_shape=jax.ShapeDtypeStruct((n, d), jnp.float32),
    )(sp, xw2, cnt, b2)


def kernel(x, edge_index, W1, b1, W2, b2):
    n, d = x.shape
    e = edge_index.shape[1]
    src = edge_index[0]
    dst = edge_index[1]

    cnt = _sc_count(dst, n=n, e=e, d=d)
    y1, xw1 = _dense1(cnt, x, W1, n=n, d=d)
    s1 = _sc_scatter(y1, src, dst, n=n, e=e, d=d)
    y2, xw2 = _dense2(s1, xw1, cnt, b1.reshape(1, d), W2, n=n, d=d)
    s2 = _sc_scatter(y2, src, dst, n=n, e=e, d=d)
    return _dense3(s2, xw2, cnt, b2.reshape(1, d), n=n, d=d)


# speedup vs baseline: 24.4840x; 1.0036x over previous
import functools

import jax
import jax.numpy as jnp
from jax import lax
from jax.experimental import pallas as pl
from jax.experimental.pallas import tpu as pltpu
from jax.experimental.pallas import tpu_sc as plsc

NC = 2
NS = 16
LANES = 16


def _pad_rows(n):
    rpt = -(-n // NS)
    rpt = -(-rpt // 128) * 128
    return rpt, rpt * NS


@functools.partial(jax.jit, static_argnames=("n", "e", "d"))
def _sc_count(dst, *, n, e, d):
    nw = NC * NS
    epw = e // nw
    blk = 128
    nblk = epw // blk
    tail = epw - nblk * blk
    rpt, n_pad = _pad_rows(n)
    zr = 32
    ob = 128

    mesh = plsc.VectorSubcoreMesh(
        core_axis_name="c", subcore_axis_name="s",
        num_cores=NC, num_subcores=NS)

    @functools.partial(
        pl.kernel,
        out_type=jax.ShapeDtypeStruct((NC, n_pad, d), jnp.float32),
        mesh=mesh,
        scratch_types=[
            pltpu.VMEM((blk,), jnp.int32),
            pltpu.VMEM((blk,), jnp.int32),
            pltpu.VMEM((max(tail, 8),), jnp.int32),
            pltpu.VMEM((blk, d), jnp.float32),
            pltpu.VMEM((zr, d), jnp.float32),
            pltpu.VMEM((ob, d), jnp.float32),
            pltpu.VMEM_SHARED((n_pad, d), jnp.float32),
            pltpu.SemaphoreType.DMA,
            pltpu.SemaphoreType.DMA,
        ],
    )
    def k(dst_hbm, out_hbm, dst_a, dst_b, dst_t, ones_v, zbuf, obuf, acc,
          isem_a, isem_b):
        c = lax.axis_index("c")
        s = lax.axis_index("s")
        wid = s * NC + c
        one = jnp.ones((LANES,), jnp.float32)
        zero = jnp.zeros((LANES,), jnp.float32)
        for r in range(blk):
            for kk in range(d // LANES):
                ones_v[r, pl.ds(kk * LANES, LANES)] = one
        for r in range(zr):
            for kk in range(d // LANES):
                zbuf[r, pl.ds(kk * LANES, LANES)] = zero
        row0 = s * rpt

        def zb(i, _):
            pltpu.sync_copy(zbuf, acc.at[pl.ds(row0 + i * zr, zr)])
            return 0

        lax.fori_loop(0, rpt // zr, zb, 0)
        plsc.subcore_barrier()

        base_e = wid * epw

        def idx_load(b, dst_n, sem_n):
            pltpu.async_copy(dst_hbm.at[pl.ds(base_e + b * blk, blk)],
                             dst_n, sem_n)

        def idx_wait(dst_n, sem_n):
            pltpu.make_async_copy(dst_hbm.at[pl.ds(base_e, blk)],
                                  dst_n, sem_n).wait()

        idx_load(0, dst_a, isem_a)

        def body(i, _):
            def stage(dst_c, isem_c, dst_n, isem_n):
                @pl.when(i + 1 < nblk)
                def _():
                    idx_load(i + 1, dst_n, isem_n)

                idx_wait(dst_c, isem_c)
                pltpu.sync_copy(ones_v, acc.at[dst_c], add=True)

            @pl.when(i % 2 == 0)
            def _():
                stage(dst_a, isem_a, dst_b, isem_b)

            @pl.when(i % 2 == 1)
            def _():
                stage(dst_b, isem_b, dst_a, isem_a)

            return 0

        lax.fori_loop(0, nblk, body, 0)

        if tail:
            e0t = base_e + nblk * blk
            pltpu.sync_copy(dst_hbm.at[pl.ds(e0t, tail)],
                            dst_t.at[pl.ds(0, tail)])
            pltpu.sync_copy(ones_v.at[pl.ds(0, tail)],
                            acc.at[dst_t], add=True)

        plsc.subcore_barrier()

        def wb(i, _):
            r0 = row0 + i * ob
            pltpu.sync_copy(acc.at[pl.ds(r0, ob)], obuf)
            pltpu.sync_copy(obuf, out_hbm.at[c].at[pl.ds(r0, ob)])
            return 0

        lax.fori_loop(0, rpt // ob, wb, 0)

    return k(dst)[:, :n, :]


@functools.partial(jax.jit, static_argnames=("n", "e", "d"))
def _sc_scatter(y, src, dst, *, n, e, d):
    nw = NC * NS
    epw = e // nw
    blk = 128
    nblk = epw // blk
    tail = epw - nblk * blk
    rpt, n_pad = _pad_rows(n)
    zr = 16
    ob = 64

    mesh = plsc.VectorSubcoreMesh(
        core_axis_name="c", subcore_axis_name="s",
        num_cores=NC, num_subcores=NS)

    @functools.partial(
        pl.kernel,
        out_type=jax.ShapeDtypeStruct((NC, n_pad, d), jnp.float32),
        mesh=mesh,
        scratch_types=[
            pltpu.VMEM((blk,), jnp.int32),
            pltpu.VMEM((blk,), jnp.int32),
            pltpu.VMEM((blk,), jnp.int32),
            pltpu.VMEM((blk,), jnp.int32),
            pltpu.VMEM((blk, d), jnp.float32),
            pltpu.VMEM((blk, d), jnp.float32),
            pltpu.VMEM((max(tail, 8),), jnp.int32),
            pltpu.VMEM((max(tail, 8),), jnp.int32),
            pltpu.VMEM((max(tail, 1), d), jnp.float32),
            pltpu.VMEM((zr, d), jnp.float32),
            pltpu.VMEM((ob, d), jnp.float32),
            pltpu.VMEM_SHARED((n_pad, d), jnp.float32),
            pltpu.SemaphoreType.DMA,
            pltpu.SemaphoreType.DMA,
            pltpu.SemaphoreType.DMA,
            pltpu.SemaphoreType.DMA,
        ],
    )
    def k(y_hbm, src_hbm, dst_hbm, out_hbm,
          src_a, dst_a, src_b, dst_b, rows_a, rows_b,
          src_t, dst_t, rows_t, zbuf, obuf, acc,
          sem_a, sem_b, isem_a, isem_b):
        c = lax.axis_index("c")
        s = lax.axis_index("s")
        wid = s * NC + c
        zero = jnp.zeros((LANES,), jnp.float32)
        for r in range(zr):
            for kk in range(d // LANES):
                zbuf[r, pl.ds(kk * LANES, LANES)] = zero
        row0 = s * rpt

        def zb(i, _):
            pltpu.sync_copy(zbuf, acc.at[pl.ds(row0 + i * zr, zr)])
            return 0

        lax.fori_loop(0, rpt // zr, zb, 0)
        plsc.subcore_barrier()

        base_e = wid * epw

        def idx_load(b, src_n, dst_n, sem_n):
            e0n = base_e + b * blk
            pltpu.async_copy(src_hbm.at[pl.ds(e0n, blk)], src_n, sem_n)
            pltpu.async_copy(dst_hbm.at[pl.ds(e0n, blk)], dst_n, sem_n)

        def idx_wait(src_n, dst_n, sem_n):
            pltpu.make_async_copy(src_hbm.at[pl.ds(base_e, blk)],
                                  src_n, sem_n).wait()
            pltpu.make_async_copy(dst_hbm.at[pl.ds(base_e, blk)],
                                  dst_n, sem_n).wait()

        idx_load(0, src_a, dst_a, isem_a)
        idx_wait(src_a, dst_a, isem_a)
        pltpu.async_copy(y_hbm.at[src_a], rows_a, sem_a)
        if nblk > 1:
            idx_load(1, src_b, dst_b, isem_b)

        def body(i, _):
            def stage(src_c, dst_c, rows_c, sem_c, isem_c,
                      src_n, dst_n, rows_n, sem_n, isem_n):
                @pl.when(i + 1 < nblk)
                def _():
                    idx_wait(src_n, dst_n, isem_n)
                    pltpu.async_copy(y_hbm.at[src_n], rows_n, sem_n)

                pltpu.make_async_copy(y_hbm.at[src_c], rows_c, sem_c).wait()
                pltpu.sync_copy(rows_c, acc.at[dst_c], add=True)

                @pl.when(i + 2 < nblk)
                def _():
                    idx_load(i + 2, src_c, dst_c, isem_c)

            @pl.when(i % 2 == 0)
            def _():
                stage(src_a, dst_a, rows_a, sem_a, isem_a,
                      src_b, dst_b, rows_b, sem_b, isem_b)

            @pl.when(i % 2 == 1)
            def _():
                stage(src_b, dst_b, rows_b, sem_b, isem_b,
                      src_a, dst_a, rows_a, sem_a, isem_a)

            return 0

        lax.fori_loop(0, nblk, body, 0)

        if tail:
            e0t = base_e + nblk * blk
            pltpu.sync_copy(src_hbm.at[pl.ds(e0t, tail)],
                            src_t.at[pl.ds(0, tail)])
            pltpu.sync_copy(dst_hbm.at[pl.ds(e0t, tail)],
                            dst_t.at[pl.ds(0, tail)])
            pltpu.async_copy(y_hbm.at[src_t], rows_t, sem_a).wait()
            pltpu.sync_copy(rows_t, acc.at[dst_t], add=True)

        plsc.subcore_barrier()

        def wb(i, _):
            r0 = row0 + i * ob
            pltpu.sync_copy(acc.at[pl.ds(r0, ob)], obuf)
            pltpu.sync_copy(obuf, out_hbm.at[c].at[pl.ds(r0, ob)])
            return 0

        lax.fori_loop(0, rpt // ob, wb, 0)

    return k(y, src, dst)[:, :n, :]


_RB = 1000


def _dinv_block(cnt_ref):
    deg = cnt_ref[0, :, 0:1] + cnt_ref[1, :, 0:1] + 1.0
    return lax.rsqrt(deg)


def _mm_body(x_ref, w_ref, xw_ref):
    xw_ref[...] = jnp.dot(x_ref[...], w_ref[...],
                          preferred_element_type=jnp.float32)


def _scale_body(cnt_ref, xw_ref, y_ref):
    y_ref[...] = xw_ref[...] * _dinv_block(cnt_ref)


def _dense2_body(sp_ref, xw1_ref, cnt_ref, b_ref, w_ref, y_ref, xw_ref):
    dinv = _dinv_block(cnt_ref)
    pre = dinv * (sp_ref[0] + sp_ref[1]) \
        + (dinv * dinv) * xw1_ref[...] + b_ref[...]
    h = jnp.maximum(pre, 0.0)
    xw = jnp.dot(h, w_ref[...], preferred_element_type=jnp.float32)
    xw_ref[...] = xw
    y_ref[...] = xw * dinv


def _dense3_body(sp_ref, xw2_ref, cnt_ref, b_ref, out_ref):
    dinv = _dinv_block(cnt_ref)
    o = dinv * (sp_ref[0] + sp_ref[1]) \
        + (dinv * dinv) * xw2_ref[...] + b_ref[...]
    m = jnp.max(o, axis=1, keepdims=True)
    lse = jnp.log(jnp.sum(jnp.exp(o - m), axis=1, keepdims=True))
    out_ref[...] = o - m - lse


def _dense_mm(x, w, *, n, d):
    grid = (n // _RB,)
    return pl.pallas_call(
        _mm_body,
        grid=grid,
        in_specs=[
            pl.BlockSpec((_RB, d), lambda i: (i, 0)),
            pl.BlockSpec((d, d), lambda i: (0, 0)),
        ],
        out_specs=pl.BlockSpec((_RB, d), lambda i: (i, 0)),
        out_shape=jax.ShapeDtypeStruct((n, d), jnp.float32),
    )(x, w)


def _dense_scale(cnt, xw, *, n, d):
    grid = (n // _RB,)
    return pl.pallas_call(
        _scale_body,
        grid=grid,
        in_specs=[
            pl.BlockSpec((NC, _RB, d), lambda i: (0, i, 0)),
            pl.BlockSpec((_RB, d), lambda i: (i, 0)),
        ],
        out_specs=pl.BlockSpec((_RB, d), lambda i: (i, 0)),
        out_shape=jax.ShapeDtypeStruct((n, d), jnp.float32),
    )(cnt, xw)


def _dense2(sp, xw1, cnt, b1, w2, *, n, d):
    grid = (n // _RB,)
    return pl.pallas_call(
        _dense2_body,
        grid=grid,
        in_specs=[
            pl.BlockSpec((NC, _RB, d), lambda i: (0, i, 0)),
            pl.BlockSpec((_RB, d), lambda i: (i, 0)),
            pl.BlockSpec((NC, _RB, d), lambda i: (0, i, 0)),
            pl.BlockSpec((1, d), lambda i: (0, 0)),
            pl.BlockSpec((d, d), lambda i: (0, 0)),
        ],
        out_specs=[
            pl.BlockSpec((_RB, d), lambda i: (i, 0)),
            pl.BlockSpec((_RB, d), lambda i: (i, 0)),
        ],
        out_shape=[
            jax.ShapeDtypeStruct((n, d), jnp.float32),
            jax.ShapeDtypeStruct((n, d), jnp.float32),
        ],
    )(sp, xw1, cnt, b1, w2)


def _dense3(sp, xw2, cnt, b2, *, n, d):
    grid = (n // _RB,)
    return pl.pallas_call(
        _dense3_body,
        grid=grid,
        in_specs=[
            pl.BlockSpec((NC, _RB, d), lambda i: (0, i, 0)),
            pl.BlockSpec((_RB, d), lambda i: (i, 0)),
            pl.BlockSpec((NC, _RB, d), lambda i: (0, i, 0)),
            pl.BlockSpec((1, d), lambda i: (0, 0)),
        ],
        out_specs=pl.BlockSpec((_RB, d), lambda i: (i, 0)),
        out_shape=jax.ShapeDtypeStruct((n, d), jnp.float32),
    )(sp, xw2, cnt, b2)


def kernel(x, edge_index, W1, b1, W2, b2):
    n, d = x.shape
    e = edge_index.shape[1]
    src = edge_index[0]
    dst = edge_index[1]

    cnt = _sc_count(dst, n=n, e=e, d=d)
    xw1 = _dense_mm(x, W1, n=n, d=d)
    y1 = _dense_scale(cnt, xw1, n=n, d=d)
    s1 = _sc_scatter(y1, src, dst, n=n, e=e, d=d)
    y2, xw2 = _dense2(s1, xw1, cnt, b1.reshape(1, d), W2, n=n, d=d)
    s2 = _sc_scatter(y2, src, dst, n=n, e=e, d=d)
    return _dense3(s2, xw2, cnt, b2.reshape(1, d), n=n, d=d)
